# Initial kernel scaffold; baseline (speedup 1.0000x reference)
import jax
import jax.numpy as jnp
from jax.experimental import pallas as pl


def kernel(x, edge_index, W_in, b_in, W_qkv, b_qkv, W_o, b_o, ln_g, ln_b, W1, att_src1, att_dst1, bias1, W2, att_src2, att_dst2, bias2, fc_W, fc_b):
    raise NotImplementedError("write your pallas kernel here")


# speedup vs baseline: 19.0070x; 19.0070x over previous
import functools

import jax
import jax.numpy as jnp
import numpy as np
from jax import lax
from jax.experimental import pallas as pl
from jax.experimental.pallas import tpu as pltpu
from jax.experimental.pallas import tpu_sc as plsc

N = 50000
T = 12
F_IN = 7
D = 32
H = 4
HF = 32
DH = D // H

NC, NS, LANES = 2, 16, 16
NW = NC * NS
E0 = 800000
CH = 128
E_PAD = 819200
N_PAD = 50048
RPT = N_PAD // NS
NEG = -1e30

NB = 400
GRID = N // NB

_f32 = jnp.float32

_REP = np.tile(np.eye(D, dtype=np.float32), (1, T))
_HM2 = np.zeros((T * D, T * H), np.float32)
for _j in range(T):
    for _h in range(H):
        _HM2[_j * D + _h * DH:_j * D + (_h + 1) * DH, _j * H + _h] = (
            1.0 / np.sqrt(DH))
_G = np.zeros((T * T * H, T * H), np.float32)
for _i in range(T):
    for _j in range(T):
        for _h in range(H):
            _G[(_i * T + _j) * H + _h, _i * H + _h] = 1.0
_EX2 = np.zeros((T * H, T * D), np.float32)
for _j in range(T):
    for _h in range(H):
        _EX2[_j * H + _h, _j * D + _h * DH:_j * D + (_h + 1) * DH] = 1.0
_SUMJ = np.tile(np.eye(D, dtype=np.float32), (T, 1))


def _dense_body(xT_ref, W_in, b_in, W_qkv, b_qkv, W_o, b_o, ln_g, ln_b,
                W1, a_src, a_dst, rep_r, hm2_r, g_r, gt_r, ex2_r, sumj_r,
                h1h_ref, ss_ref, sd_ref):
    rep = rep_r[...]
    hm2 = hm2_r[...]
    g = g_r[...]
    gt = gt_r[...]
    ex2 = ex2_r[...]
    sumj = sumj_r[...]

    xs_t, q_t, k_t, v_t = [], [], [], []
    for t in range(T):
        xs = jnp.dot(xT_ref[t], W_in[...], preferred_element_type=_f32) \
            + b_in[...]
        xs_t.append(xs)
        qkv = lax.dot_general(xs, W_qkv[...], (((1,), (1,)), ((), ())),
                              preferred_element_type=_f32) + b_qkv[...]
        q_t.append(qkv[:, :D])
        k_t.append(qkv[:, D:2 * D])
        v_t.append(qkv[:, 2 * D:])
    K_all = jnp.concatenate(k_t, axis=1)
    V_all = jnp.concatenate(v_t, axis=1)

    sc_rows = []
    for i in range(T):
        qrep = jnp.dot(q_t[i], rep, preferred_element_type=_f32)
        sc_rows.append(jnp.dot(qrep * K_all, hm2,
                               preferred_element_type=_f32))
    SCw = jnp.concatenate(sc_rows, axis=1)
    Ew = jnp.exp(SCw)
    den = jnp.dot(Ew, g, preferred_element_type=_f32)
    den_b = jnp.dot(den, gt, preferred_element_type=_f32)
    Aw = Ew / den_b

    hacc = jnp.zeros((NB, D), _f32)
    for i in range(T):
        Ai = jnp.dot(Aw[:, i * T * H:(i + 1) * T * H], ex2,
                     preferred_element_type=_f32)
        att = jnp.dot(Ai * V_all, sumj, preferred_element_type=_f32)
        r = lax.dot_general(att, W_o[...], (((1,), (1,)), ((), ())),
                            preferred_element_type=_f32) + b_o[...] + xs_t[i]
        mu = r.mean(axis=-1, keepdims=True)
        var = ((r - mu) ** 2).mean(axis=-1, keepdims=True)
        hacc = hacc + (r - mu) * lax.rsqrt(var + 1e-5) * ln_g[...] + ln_b[...]
    hfeat = hacc / T

    hs, sss, sds = [], [], []
    for h in range(H):
        hh = jnp.dot(hfeat, W1[...][:, h * HF:(h + 1) * HF],
                     preferred_element_type=_f32)
        hs.append(hh)
        sss.append((hh * a_src[...][h]).sum(-1))
        sds.append((hh * a_dst[...][h]).sum(-1))
    z = [jnp.zeros((NB,), _f32)] * (16 - H)
    h1h_ref[...] = jnp.stack(hs, axis=0)
    ss_ref[...] = jnp.stack(sss + z, axis=1)
    sd_ref[...] = jnp.stack(sds + z, axis=1)


def _full(shape):
    nd = len(shape)
    return pl.BlockSpec(shape, lambda i: (0,) * nd)


_dense_call = pl.pallas_call(
    _dense_body,
    grid=(GRID,),
    in_specs=[
        pl.BlockSpec((T, NB, F_IN), lambda i: (0, i, 0)),
        _full((F_IN, D)), _full((D,)), _full((3 * D, D)), _full((3 * D,)),
        _full((D, D)), _full((D,)), _full((D,)), _full((D,)),
        _full((D, H * HF)), _full((H, HF)), _full((H, HF)),
        _full((D, T * D)), _full((T * D, T * H)), _full((T * T * H, T * H)),
        _full((T * H, T * T * H)), _full((T * H, T * D)), _full((T * D, D)),
    ],
    out_specs=[
        pl.BlockSpec((H, NB, HF), lambda i: (0, i, 0)),
        pl.BlockSpec((NB, 16), lambda i: (i, 0)),
        pl.BlockSpec((NB, 16), lambda i: (i, 0)),
    ],
    out_shape=[
        jax.ShapeDtypeStruct((H, N, HF), _f32),
        jax.ShapeDtypeStruct((N, 16), _f32),
        jax.ShapeDtypeStruct((N, 16), _f32),
    ],
)


def _mid_body(aggh_ref, den_ref, bias, W2, a_src, a_dst,
              h2h_ref, ss_ref, sd_ref):
    acc = jnp.zeros((NB, HF), _f32)
    for h in range(H):
        d = den_ref[...][:, h].reshape(NB, 1)
        acc = acc + aggh_ref[...][h] / (d + 1e-16)
    hfeat = jnp.maximum(acc / H + bias[...], 0.0)
    hs, sss, sds = [], [], []
    for h in range(H):
        hh = jnp.dot(hfeat, W2[...][:, h * HF:(h + 1) * HF],
                     preferred_element_type=_f32)
        hs.append(hh)
        sss.append((hh * a_src[...][h]).sum(-1))
        sds.append((hh * a_dst[...][h]).sum(-1))
    z = [jnp.zeros((NB,), _f32)] * (16 - H)
    h2h_ref[...] = jnp.stack(hs, axis=0)
    ss_ref[...] = jnp.stack(sss + z, axis=1)
    sd_ref[...] = jnp.stack(sds + z, axis=1)


_mid_call = pl.pallas_call(
    _mid_body,
    grid=(GRID,),
    in_specs=[
        pl.BlockSpec((H, NB, HF), lambda i: (0, i, 0)),
        pl.BlockSpec((NB, H), lambda i: (i, 0)),
        _full((HF,)), _full((HF, H * HF)), _full((H, HF)), _full((H, HF)),
    ],
    out_specs=[
        pl.BlockSpec((H, NB, HF), lambda i: (0, i, 0)),
        pl.BlockSpec((NB, 16), lambda i: (i, 0)),
        pl.BlockSpec((NB, 16), lambda i: (i, 0)),
    ],
    out_shape=[
        jax.ShapeDtypeStruct((H, N, HF), _f32),
        jax.ShapeDtypeStruct((N, 16), _f32),
        jax.ShapeDtypeStruct((N, 16), _f32),
    ],
)


def _final_body(aggh_ref, den_ref, bias, fc_W, fc_b, out_ref):
    acc = jnp.zeros((NB, HF), _f32)
    for h in range(H):
        d = den_ref[...][:, h].reshape(NB, 1)
        acc = acc + aggh_ref[...][h] / (d + 1e-16)
    hfeat = jnp.maximum(acc / H + bias[...], 0.0)
    out_ref[...] = jnp.dot(hfeat, fc_W[...],
                           preferred_element_type=_f32) + fc_b[...]


_final_call = pl.pallas_call(
    _final_body,
    grid=(GRID,),
    in_specs=[
        pl.BlockSpec((H, NB, HF), lambda i: (0, i, 0)),
        pl.BlockSpec((NB, H), lambda i: (i, 0)),
        _full((HF,)), _full((HF, 1)), _full((1,)),
    ],
    out_specs=pl.BlockSpec((NB, 1), lambda i: (i, 0)),
    out_shape=jax.ShapeDtypeStruct((N, 1), _f32),
)


BE = 6400


def _etrans_body(sel_r, ein_r, eout_r):
    eout_r[...] = lax.dot_general(sel_r[...], ein_r[...],
                                  (((1,), (1,)), ((), ())),
                                  preferred_element_type=_f32)


_etrans_call = pl.pallas_call(
    _etrans_body,
    grid=(E_PAD // BE,),
    in_specs=[
        _full((H, 16)),
        pl.BlockSpec((BE, 16), lambda i: (i, 0)),
    ],
    out_specs=pl.BlockSpec((H, BE), lambda i: (0, i)),
    out_shape=jax.ShapeDtypeStruct((H, E_PAD), _f32),
)

_SEL = np.eye(H, 16, dtype=np.float32)


def _escore_body(srcp, dstp, ss, sd, e1, sidx, didx, ssr, sdr, ew,
                 sem_a, sem_b):
    c = lax.axis_index("c")
    s = lax.axis_index("s")
    wid = s * NC + c
    cpt = E_PAD // (NW * CH)

    def chunk(i, carry):
        base = (wid * cpt + i) * CH
        pltpu.sync_copy(srcp.at[pl.ds(base, CH)], sidx)
        pltpu.sync_copy(dstp.at[pl.ds(base, CH)], didx)
        cp1 = pltpu.async_copy(ss.at[sidx], ssr, sem_a)
        cp2 = pltpu.async_copy(sd.at[didx], sdr, sem_b)
        cp1.wait()
        cp2.wait()

        def vec(j, carry2):
            al = ssr[j, pl.ds(0, LANES)] + sdr[j, pl.ds(0, LANES)]
            al = jnp.where(al > 0, al, 0.2 * al)
            ew[j, pl.ds(0, LANES)] = jnp.exp(al)
            return carry2

        lax.fori_loop(0, CH, vec, 0, unroll=8)
        pltpu.sync_copy(ew, e1.at[pl.ds(base, CH)])
        return carry

    lax.fori_loop(0, cpt, chunk, 0)


@functools.cache
def _get_escore_call():
  return pl.kernel(
    _escore_body,
    out_type=jax.ShapeDtypeStruct((E_PAD, 16), _f32),
    mesh=plsc.VectorSubcoreMesh(core_axis_name="c", subcore_axis_name="s",
                                num_cores=NC, num_subcores=NS),
    scratch_types=[
        pltpu.VMEM((CH,), jnp.int32),
        pltpu.VMEM((CH,), jnp.int32),
        pltpu.VMEM((CH, 16), _f32),
        pltpu.VMEM((CH, 16), _f32),
        pltpu.VMEM((CH, 16), _f32),
        pltpu.SemaphoreType.DMA,
        pltpu.SemaphoreType.DMA,
    ],
    compiler_params=pltpu.CompilerParams(use_tc_tiling_on_sc=False),
  )


def _agg_body(srcp, dstp, e1, h1f, z2d, z1d, aggh, denp,
              sidx, didx, ech, gidx, rowsb, numacc, denacc, sem):
    c = lax.axis_index("c")
    s = lax.axis_index("s")
    cpt = E_PAD // (NS * CH)
    for r in range(2):
        q = 2 * r + c
        pltpu.sync_copy(z2d, numacc.at[pl.ds(s * RPT, RPT)])
        pltpu.sync_copy(z1d.at[pl.ds(s * RPT, RPT)],
                        denacc.at[pl.ds(s * RPT, RPT)])
        plsc.subcore_barrier()

        qoff = q * N_PAD

        def chunk(i, carry):
            base = (s * cpt + i) * CH
            pltpu.sync_copy(srcp.at[pl.ds(base, CH)], sidx)
            pltpu.sync_copy(dstp.at[pl.ds(base, CH)], didx)
            pltpu.sync_copy(e1.at[q, pl.ds(base, CH)], ech.at[pl.ds(0, CH)])
            for v in range(CH // LANES):
                gidx[pl.ds(v * LANES, LANES)] = (
                    sidx[pl.ds(v * LANES, LANES)] + qoff)
            pltpu.async_copy(h1f.at[gidx], rowsb, sem).wait()

            def scl(j, carry2):
                ev = ech[pl.ds(j, LANES)]
                e = ev[0]
                a = rowsb[j, pl.ds(0, LANES)]
                rowsb[j, pl.ds(0, LANES)] = a * e
                b = rowsb[j, pl.ds(LANES, LANES)]
                rowsb[j, pl.ds(LANES, LANES)] = b * e
                return carry2

            lax.fori_loop(0, CH, scl, 0, unroll=8)
            pltpu.sync_copy(ech.at[pl.ds(0, CH)], denacc.at[didx], add=True)
            pltpu.sync_copy(rowsb, numacc.at[didx], add=True)
            return carry

        lax.fori_loop(0, cpt, chunk, 0)
        plsc.subcore_barrier()
        pltpu.sync_copy(numacc.at[pl.ds(s * RPT, RPT)],
                        aggh.at[q, pl.ds(s * RPT, RPT)])
        pltpu.sync_copy(denacc.at[pl.ds(s * RPT, RPT)],
                        denp.at[q, pl.ds(s * RPT, RPT)])
        plsc.subcore_barrier()


@functools.cache
def _get_agg_call():
  return pl.kernel(
    _agg_body,
    out_type=[
        jax.ShapeDtypeStruct((H, N_PAD, HF), _f32),
        jax.ShapeDtypeStruct((H, N_PAD), _f32),
    ],
    mesh=plsc.VectorSubcoreMesh(core_axis_name="c", subcore_axis_name="s",
                                num_cores=NC, num_subcores=NS),
    scratch_types=[
        pltpu.VMEM((CH,), jnp.int32),
        pltpu.VMEM((CH,), jnp.int32),
        pltpu.VMEM((CH + LANES,), _f32),
        pltpu.VMEM((CH,), jnp.int32),
        pltpu.VMEM((CH, HF), _f32),
        pltpu.VMEM_SHARED((N_PAD, HF), _f32),
        pltpu.VMEM_SHARED((N_PAD,), _f32),
        pltpu.SemaphoreType.DMA,
    ],
    compiler_params=pltpu.CompilerParams(use_tc_tiling_on_sc=False),
  )


def kernel(x, edge_index, W_in, b_in, W_qkv, b_qkv, W_o, b_o, ln_g, ln_b,
           W1, att_src1, att_dst1, bias1, W2, att_src2, att_dst2, bias2,
           fc_W, fc_b):
    src = edge_index[0].astype(jnp.int32)
    dst = edge_index[1].astype(jnp.int32)
    pad = jnp.full((E_PAD - E0,), N, jnp.int32)
    srcp = jnp.concatenate([src, pad])
    dstp = jnp.concatenate([dst, pad])

    xT = x.reshape(N, T, F_IN).transpose(1, 0, 2)
    h1h, ss1, sd1 = _dense_call(xT, W_in, b_in, W_qkv, b_qkv, W_o, b_o,
                                ln_g, ln_b, W1, att_src1, att_dst1,
                                jnp.asarray(_REP), jnp.asarray(_HM2),
                                jnp.asarray(_G), jnp.asarray(_G.T),
                                jnp.asarray(_EX2), jnp.asarray(_SUMJ))

    z2d = jnp.zeros((RPT, HF), _f32)
    z1d = jnp.zeros((N_PAD,), _f32)

    def spad(a):
        return jnp.concatenate([a, jnp.full((N_PAD - N, 16), NEG, _f32)])

    def hpad(hh):
        return jnp.concatenate(
            [hh, jnp.zeros((H, N_PAD - N, HF), _f32)], axis=1
        ).reshape(H * N_PAD, HF)

    escore = _get_escore_call()
    agg = _get_agg_call()
    sel = jnp.asarray(_SEL)
    e1 = _etrans_call(sel, escore(srcp, dstp, spad(ss1), spad(sd1)))
    agg1, den1 = agg(srcp, dstp, e1, hpad(h1h), z2d, z1d)

    h2h, ss2, sd2 = _mid_call(agg1[:, :N], den1[:, :N].T, bias1, W2,
                              att_src2, att_dst2)
    e2 = _etrans_call(sel, escore(srcp, dstp, spad(ss2), spad(sd2)))
    agg2, den2 = agg(srcp, dstp, e2, hpad(h2h), z2d, z1d)

    return _final_call(agg2[:, :N], den2[:, :N].T, bias2, fc_W, fc_b)


# speedup vs baseline: 25.0032x; 1.3155x over previous
import functools

import jax
import jax.numpy as jnp
import numpy as np
from jax import lax
from jax.experimental import pallas as pl
from jax.experimental.pallas import tpu as pltpu
from jax.experimental.pallas import tpu_sc as plsc

N = 50000
T = 12
F_IN = 7
D = 32
H = 4
HF = 32
DH = D // H

NC, NS, LANES = 2, 16, 16
NW = NC * NS
E0 = 800000
CH = 128
E_PAD = 819200
N_PAD = 50048
RPT = N_PAD // NS
NEG = -1e30

NB = 400
GRID = N // NB

_f32 = jnp.float32

_REP = np.tile(np.eye(D, dtype=np.float32), (1, T))
_HM2 = np.zeros((T * D, T * H), np.float32)
for _j in range(T):
    for _h in range(H):
        _HM2[_j * D + _h * DH:_j * D + (_h + 1) * DH, _j * H + _h] = (
            1.0 / np.sqrt(DH))
_G = np.zeros((T * T * H, T * H), np.float32)
for _i in range(T):
    for _j in range(T):
        for _h in range(H):
            _G[(_i * T + _j) * H + _h, _i * H + _h] = 1.0
_EX2 = np.zeros((T * H, T * D), np.float32)
for _j in range(T):
    for _h in range(H):
        _EX2[_j * H + _h, _j * D + _h * DH:_j * D + (_h + 1) * DH] = 1.0
_SUMJ = np.tile(np.eye(D, dtype=np.float32), (T, 1))


def _dense_body(xT_ref, W_in, b_in, W_qkv, b_qkv, W_o, b_o, ln_g, ln_b,
                W1, a_src, a_dst, rep_r, hm2_r, g_r, gt_r, ex2_r, sumj_r,
                h1h_ref, ss_ref, sd_ref):
    rep = rep_r[...]
    hm2 = hm2_r[...]
    g = g_r[...]
    gt = gt_r[...]
    ex2 = ex2_r[...]
    sumj = sumj_r[...]

    xs_t, q_t, k_t, v_t = [], [], [], []
    for t in range(T):
        xs = jnp.dot(xT_ref[t], W_in[...], preferred_element_type=_f32) \
            + b_in[...]
        xs_t.append(xs)
        qkv = lax.dot_general(xs, W_qkv[...], (((1,), (1,)), ((), ())),
                              preferred_element_type=_f32) + b_qkv[...]
        q_t.append(qkv[:, :D])
        k_t.append(qkv[:, D:2 * D])
        v_t.append(qkv[:, 2 * D:])
    K_all = jnp.concatenate(k_t, axis=1)
    V_all = jnp.concatenate(v_t, axis=1)

    sc_rows = []
    for i in range(T):
        qrep = jnp.dot(q_t[i], rep, preferred_element_type=_f32)
        sc_rows.append(jnp.dot(qrep * K_all, hm2,
                               preferred_element_type=_f32))
    SCw = jnp.concatenate(sc_rows, axis=1)
    Ew = jnp.exp(SCw)
    den = jnp.dot(Ew, g, preferred_element_type=_f32)
    den_b = jnp.dot(den, gt, preferred_element_type=_f32)
    Aw = Ew / den_b

    hacc = jnp.zeros((NB, D), _f32)
    for i in range(T):
        Ai = jnp.dot(Aw[:, i * T * H:(i + 1) * T * H], ex2,
                     preferred_element_type=_f32)
        att = jnp.dot(Ai * V_all, sumj, preferred_element_type=_f32)
        r = lax.dot_general(att, W_o[...], (((1,), (1,)), ((), ())),
                            preferred_element_type=_f32) + b_o[...] + xs_t[i]
        mu = r.mean(axis=-1, keepdims=True)
        var = ((r - mu) ** 2).mean(axis=-1, keepdims=True)
        hacc = hacc + (r - mu) * lax.rsqrt(var + 1e-5) * ln_g[...] + ln_b[...]
    hfeat = hacc / T

    hs, sss, sds = [], [], []
    for h in range(H):
        hh = jnp.dot(hfeat, W1[...][:, h * HF:(h + 1) * HF],
                     preferred_element_type=_f32)
        hs.append(hh)
        sss.append((hh * a_src[...][h]).sum(-1))
        sds.append((hh * a_dst[...][h]).sum(-1))
    z = [jnp.zeros((NB,), _f32)] * (16 - H)
    h1h_ref[...] = jnp.stack(hs, axis=0)
    ss_ref[...] = jnp.stack(sss + z, axis=1)
    sd_ref[...] = jnp.stack(sds + z, axis=1)


def _full(shape):
    nd = len(shape)
    return pl.BlockSpec(shape, lambda i: (0,) * nd)


_dense_call = pl.pallas_call(
    _dense_body,
    grid=(GRID,),
    in_specs=[
        pl.BlockSpec((T, NB, F_IN), lambda i: (0, i, 0)),
        _full((F_IN, D)), _full((D,)), _full((3 * D, D)), _full((3 * D,)),
        _full((D, D)), _full((D,)), _full((D,)), _full((D,)),
        _full((D, H * HF)), _full((H, HF)), _full((H, HF)),
        _full((D, T * D)), _full((T * D, T * H)), _full((T * T * H, T * H)),
        _full((T * H, T * T * H)), _full((T * H, T * D)), _full((T * D, D)),
    ],
    out_specs=[
        pl.BlockSpec((H, NB, HF), lambda i: (0, i, 0)),
        pl.BlockSpec((NB, 16), lambda i: (i, 0)),
        pl.BlockSpec((NB, 16), lambda i: (i, 0)),
    ],
    out_shape=[
        jax.ShapeDtypeStruct((H, N, HF), _f32),
        jax.ShapeDtypeStruct((N, 16), _f32),
        jax.ShapeDtypeStruct((N, 16), _f32),
    ],
)


def _mid_body(aggh_ref, den_ref, bias, W2, a_src, a_dst,
              h2h_ref, ss_ref, sd_ref):
    acc = jnp.zeros((NB, HF), _f32)
    for h in range(H):
        d = den_ref[...][:, h].reshape(NB, 1)
        acc = acc + aggh_ref[...][h] / (d + 1e-16)
    hfeat = jnp.maximum(acc / H + bias[...], 0.0)
    hs, sss, sds = [], [], []
    for h in range(H):
        hh = jnp.dot(hfeat, W2[...][:, h * HF:(h + 1) * HF],
                     preferred_element_type=_f32)
        hs.append(hh)
        sss.append((hh * a_src[...][h]).sum(-1))
        sds.append((hh * a_dst[...][h]).sum(-1))
    z = [jnp.zeros((NB,), _f32)] * (16 - H)
    h2h_ref[...] = jnp.stack(hs, axis=0)
    ss_ref[...] = jnp.stack(sss + z, axis=1)
    sd_ref[...] = jnp.stack(sds + z, axis=1)


_mid_call = pl.pallas_call(
    _mid_body,
    grid=(GRID,),
    in_specs=[
        pl.BlockSpec((H, NB, HF), lambda i: (0, i, 0)),
        pl.BlockSpec((NB, H), lambda i: (i, 0)),
        _full((HF,)), _full((HF, H * HF)), _full((H, HF)), _full((H, HF)),
    ],
    out_specs=[
        pl.BlockSpec((H, NB, HF), lambda i: (0, i, 0)),
        pl.BlockSpec((NB, 16), lambda i: (i, 0)),
        pl.BlockSpec((NB, 16), lambda i: (i, 0)),
    ],
    out_shape=[
        jax.ShapeDtypeStruct((H, N, HF), _f32),
        jax.ShapeDtypeStruct((N, 16), _f32),
        jax.ShapeDtypeStruct((N, 16), _f32),
    ],
)


def _final_body(aggh_ref, den_ref, bias, fc_W, fc_b, out_ref):
    acc = jnp.zeros((NB, HF), _f32)
    for h in range(H):
        d = den_ref[...][:, h].reshape(NB, 1)
        acc = acc + aggh_ref[...][h] / (d + 1e-16)
    hfeat = jnp.maximum(acc / H + bias[...], 0.0)
    out_ref[...] = jnp.dot(hfeat, fc_W[...],
                           preferred_element_type=_f32) + fc_b[...]


_final_call = pl.pallas_call(
    _final_body,
    grid=(GRID,),
    in_specs=[
        pl.BlockSpec((H, NB, HF), lambda i: (0, i, 0)),
        pl.BlockSpec((NB, H), lambda i: (i, 0)),
        _full((HF,)), _full((HF, 1)), _full((1,)),
    ],
    out_specs=pl.BlockSpec((NB, 1), lambda i: (i, 0)),
    out_shape=jax.ShapeDtypeStruct((N, 1), _f32),
)


BE = 6400


def _etrans_body(sel_r, ein_r, eout_r):
    eout_r[...] = lax.dot_general(sel_r[...], ein_r[...],
                                  (((1,), (1,)), ((), ())),
                                  preferred_element_type=_f32)


_etrans_call = pl.pallas_call(
    _etrans_body,
    grid=(E_PAD // BE,),
    in_specs=[
        _full((H, 16)),
        pl.BlockSpec((BE, 16), lambda i: (i, 0)),
    ],
    out_specs=pl.BlockSpec((H, BE), lambda i: (0, i)),
    out_shape=jax.ShapeDtypeStruct((H, E_PAD), _f32),
)

_SEL = np.eye(H, 16, dtype=np.float32)


def _escore_body(srcp, dstp, ss, sd, e1, sidx, didx, ssr, sdr, ew,
                 sem_a, sem_b):
    c = lax.axis_index("c")
    s = lax.axis_index("s")
    wid = s * NC + c
    cpt = E_PAD // (NW * CH)

    def chunk(i, carry):
        base = (wid * cpt + i) * CH
        pltpu.sync_copy(srcp.at[pl.ds(base, CH)], sidx)
        pltpu.sync_copy(dstp.at[pl.ds(base, CH)], didx)
        cp1 = pltpu.async_copy(ss.at[sidx], ssr, sem_a)
        cp2 = pltpu.async_copy(sd.at[didx], sdr, sem_b)
        cp1.wait()
        cp2.wait()

        def vec(j, carry2):
            al = ssr[j, pl.ds(0, LANES)] + sdr[j, pl.ds(0, LANES)]
            al = jnp.where(al > 0, al, 0.2 * al)
            ew[j, pl.ds(0, LANES)] = jnp.exp(al)
            return carry2

        lax.fori_loop(0, CH, vec, 0, unroll=8)
        pltpu.sync_copy(ew, e1.at[pl.ds(base, CH)])
        return carry

    lax.fori_loop(0, cpt, chunk, 0)


@functools.cache
def _get_escore_call():
  return pl.kernel(
    _escore_body,
    out_type=jax.ShapeDtypeStruct((E_PAD, 16), _f32),
    mesh=plsc.VectorSubcoreMesh(core_axis_name="c", subcore_axis_name="s",
                                num_cores=NC, num_subcores=NS),
    scratch_types=[
        pltpu.VMEM((CH,), jnp.int32),
        pltpu.VMEM((CH,), jnp.int32),
        pltpu.VMEM((CH, 16), _f32),
        pltpu.VMEM((CH, 16), _f32),
        pltpu.VMEM((CH, 16), _f32),
        pltpu.SemaphoreType.DMA,
        pltpu.SemaphoreType.DMA,
    ],
    compiler_params=pltpu.CompilerParams(use_tc_tiling_on_sc=False),
  )


def _agg_body(srcp, dstp, e1, h1f, z2d, z1d, aggh, denp,
              sidx, didx, ech, gidx, rowsb, numacc, denacc,
              sl0, sl1, sg0, sg1, sn0, sn1, sd0, sd1):
    c = lax.axis_index("c")
    s = lax.axis_index("s")
    cpt = E_PAD // (NS * CH)
    sem_l = (sl0, sl1)
    sem_g = (sg0, sg1)
    sem_n = (sn0, sn1)
    sem_d = (sd0, sd1)

    for r in range(2):
        q = 2 * r + c
        qoff = q * N_PAD
        pltpu.sync_copy(z2d.at[pl.ds(0, RPT)], numacc.at[pl.ds(s * RPT, RPT)])
        pltpu.sync_copy(z1d.at[pl.ds(s * RPT, RPT)],
                        denacc.at[pl.ds(s * RPT, RPT)])
        plsc.subcore_barrier()

        def issue_lin(i, b):
            base = (s * cpt + i) * CH
            pltpu.async_copy(srcp.at[pl.ds(base, CH)], sidx.at[b], sem_l[b])
            pltpu.async_copy(dstp.at[pl.ds(base, CH)], didx.at[b], sem_l[b])
            pltpu.async_copy(e1.at[q, pl.ds(base, CH)],
                             ech.at[b, pl.ds(0, CH)], sem_l[b])

        def wait_lin(b):
            pltpu.make_async_copy(srcp.at[pl.ds(0, CH)], sidx.at[b],
                                  sem_l[b]).wait()
            pltpu.make_async_copy(dstp.at[pl.ds(0, CH)], didx.at[b],
                                  sem_l[b]).wait()
            pltpu.make_async_copy(e1.at[0, pl.ds(0, CH)],
                                  ech.at[b, pl.ds(0, CH)], sem_l[b]).wait()

        def prep_gather(b):
            for v in range(CH // LANES):
                gidx[b, pl.ds(v * LANES, LANES)] = (
                    sidx[b, pl.ds(v * LANES, LANES)] + qoff)
            pltpu.async_copy(h1f.at[gidx.at[b]], rowsb.at[b], sem_g[b])

        def wait_gather(b):
            pltpu.make_async_copy(h1f.at[gidx.at[b]], rowsb.at[b],
                                  sem_g[b]).wait()

        def scale_scatter(b):
            def scl(j, carry2):
                ev = ech[b, pl.ds(j, LANES)]
                e = ev[0]
                a = rowsb[b, j, pl.ds(0, LANES)]
                rowsb[b, j, pl.ds(0, LANES)] = a * e
                a2 = rowsb[b, j, pl.ds(LANES, LANES)]
                rowsb[b, j, pl.ds(LANES, LANES)] = a2 * e
                return carry2

            lax.fori_loop(0, CH, scl, 0, unroll=8)
            pltpu.async_copy(ech.at[b, pl.ds(0, CH)], denacc.at[didx.at[b]],
                             sem_d[b], add=True)
            pltpu.async_copy(rowsb.at[b], numacc.at[didx.at[b]],
                             sem_n[b], add=True)

        def drain_scat(b):
            pltpu.make_async_copy(ech.at[b, pl.ds(0, CH)],
                                  denacc.at[didx.at[b]], sem_d[b]).wait()
            pltpu.make_async_copy(rowsb.at[b], numacc.at[didx.at[b]],
                                  sem_n[b]).wait()

        issue_lin(0, 0)
        wait_lin(0)
        prep_gather(0)
        pltpu.async_copy(srcp.at[pl.ds(s * cpt * CH, CH)], sidx.at[1],
                         sem_l[1])
        pltpu.async_copy(dstp.at[pl.ds(s * cpt * CH, CH)], didx.at[1],
                         sem_l[1])
        pltpu.make_async_copy(srcp.at[pl.ds(0, CH)], sidx.at[1],
                              sem_l[1]).wait()
        pltpu.make_async_copy(dstp.at[pl.ds(0, CH)], didx.at[1],
                              sem_l[1]).wait()
        pltpu.sync_copy(z2d.at[pl.ds(0, CH)], rowsb.at[1])
        pltpu.sync_copy(z1d.at[pl.ds(0, CH)], ech.at[1, pl.ds(0, CH)])
        pltpu.async_copy(ech.at[1, pl.ds(0, CH)], denacc.at[didx.at[1]],
                         sem_d[1], add=True)
        pltpu.async_copy(rowsb.at[1], numacc.at[didx.at[1]],
                         sem_n[1], add=True)

        def body(g, carry):
            i1 = 2 * g + 1
            inext = jnp.minimum(2 * g + 2, cpt - 1)
            drain_scat(1)
            issue_lin(i1, 1)
            wait_gather(0)
            scale_scatter(0)
            wait_lin(1)
            prep_gather(1)
            drain_scat(0)
            issue_lin(inext, 0)
            wait_lin(0)
            wait_gather(1)
            scale_scatter(1)
            prep_gather(0)
            return carry

        lax.fori_loop(0, cpt // 2, body, 0)
        wait_gather(0)
        drain_scat(1)
        plsc.subcore_barrier()
        pltpu.sync_copy(numacc.at[pl.ds(s * RPT, RPT)],
                        aggh.at[q, pl.ds(s * RPT, RPT)])
        pltpu.sync_copy(denacc.at[pl.ds(s * RPT, RPT)],
                        denp.at[q, pl.ds(s * RPT, RPT)])
        plsc.subcore_barrier()


@functools.cache
def _get_agg_call():
  return pl.kernel(
    _agg_body,
    out_type=[
        jax.ShapeDtypeStruct((H, N_PAD, HF), _f32),
        jax.ShapeDtypeStruct((H, N_PAD), _f32),
    ],
    mesh=plsc.VectorSubcoreMesh(core_axis_name="c", subcore_axis_name="s",
                                num_cores=NC, num_subcores=NS),
    scratch_types=[
        pltpu.VMEM((2, CH), jnp.int32),
        pltpu.VMEM((2, CH), jnp.int32),
        pltpu.VMEM((2, CH + LANES), _f32),
        pltpu.VMEM((2, CH), jnp.int32),
        pltpu.VMEM((2, CH, HF), _f32),
        pltpu.VMEM_SHARED((N_PAD, HF), _f32),
        pltpu.VMEM_SHARED((N_PAD,), _f32),
        pltpu.SemaphoreType.DMA,
        pltpu.SemaphoreType.DMA,
        pltpu.SemaphoreType.DMA,
        pltpu.SemaphoreType.DMA,
        pltpu.SemaphoreType.DMA,
        pltpu.SemaphoreType.DMA,
        pltpu.SemaphoreType.DMA,
        pltpu.SemaphoreType.DMA,
    ],
    compiler_params=pltpu.CompilerParams(use_tc_tiling_on_sc=False),
  )


def kernel(x, edge_index, W_in, b_in, W_qkv, b_qkv, W_o, b_o, ln_g, ln_b,
           W1, att_src1, att_dst1, bias1, W2, att_src2, att_dst2, bias2,
           fc_W, fc_b):
    src = edge_index[0].astype(jnp.int32)
    dst = edge_index[1].astype(jnp.int32)
    pad = jnp.full((E_PAD - E0,), N, jnp.int32)
    srcp = jnp.concatenate([src, pad])
    dstp = jnp.concatenate([dst, pad])

    xT = x.reshape(N, T, F_IN).transpose(1, 0, 2)
    h1h, ss1, sd1 = _dense_call(xT, W_in, b_in, W_qkv, b_qkv, W_o, b_o,
                                ln_g, ln_b, W1, att_src1, att_dst1,
                                jnp.asarray(_REP), jnp.asarray(_HM2),
                                jnp.asarray(_G), jnp.asarray(_G.T),
                                jnp.asarray(_EX2), jnp.asarray(_SUMJ))

    z2d = jnp.zeros((RPT, HF), _f32)
    z1d = jnp.zeros((N_PAD,), _f32)

    def spad(a):
        return jnp.concatenate([a, jnp.full((N_PAD - N, 16), NEG, _f32)])

    def hpad(hh):
        return jnp.concatenate(
            [hh, jnp.zeros((H, N_PAD - N, HF), _f32)], axis=1
        ).reshape(H * N_PAD, HF)

    escore = _get_escore_call()
    agg = _get_agg_call()
    sel = jnp.asarray(_SEL)
    e1 = _etrans_call(sel, escore(srcp, dstp, spad(ss1), spad(sd1)))
    agg1, den1 = agg(srcp, dstp, e1, hpad(h1h), z2d, z1d)

    h2h, ss2, sd2 = _mid_call(agg1[:, :N], den1[:, :N].T, bias1, W2,
                              att_src2, att_dst2)
    e2 = _etrans_call(sel, escore(srcp, dstp, spad(ss2), spad(sd2)))
    agg2, den2 = agg(srcp, dstp, e2, hpad(h2h), z2d, z1d)

    return _final_call(agg2[:, :N], den2[:, :N].T, bias2, fc_W, fc_b)


# speedup vs baseline: 26.2588x; 1.0502x over previous
import functools

import jax
import jax.numpy as jnp
import numpy as np
from jax import lax
from jax.experimental import pallas as pl
from jax.experimental.pallas import tpu as pltpu
from jax.experimental.pallas import tpu_sc as plsc

N = 50000
T = 12
F_IN = 7
D = 32
H = 4
HF = 32
DH = D // H

NC, NS, LANES = 2, 16, 16
NW = NC * NS
E0 = 800000
CH = 128
E_PAD = 819200
N_PAD = 50048
RPT = N_PAD // NS
NEG = -1e30

NB = 400
GRID = N // NB

_f32 = jnp.float32

_REP = np.tile(np.eye(D, dtype=np.float32), (1, T))
_HM2 = np.zeros((T * D, T * H), np.float32)
for _j in range(T):
    for _h in range(H):
        _HM2[_j * D + _h * DH:_j * D + (_h + 1) * DH, _j * H + _h] = (
            1.0 / np.sqrt(DH))
_G = np.zeros((T * T * H, T * H), np.float32)
for _i in range(T):
    for _j in range(T):
        for _h in range(H):
            _G[(_i * T + _j) * H + _h, _i * H + _h] = 1.0
_EX2 = np.zeros((T * H, T * D), np.float32)
for _j in range(T):
    for _h in range(H):
        _EX2[_j * H + _h, _j * D + _h * DH:_j * D + (_h + 1) * DH] = 1.0
_SUMJ = np.tile(np.eye(D, dtype=np.float32), (T, 1))


def _dense_body(xT_ref, W_in, b_in, W_qkv, b_qkv, W_o, b_o, ln_g, ln_b,
                W1, a_src, a_dst, rep_r, hm2_r, g_r, gt_r, ex2_r, sumj_r,
                h1h_ref, ss_ref, sd_ref):
    rep = rep_r[...]
    hm2 = hm2_r[...]
    g = g_r[...]
    gt = gt_r[...]
    ex2 = ex2_r[...]
    sumj = sumj_r[...]

    xs_t, q_t, k_t, v_t = [], [], [], []
    for t in range(T):
        xs = jnp.dot(xT_ref[t], W_in[...], preferred_element_type=_f32) \
            + b_in[...]
        xs_t.append(xs)
        qkv = lax.dot_general(xs, W_qkv[...], (((1,), (1,)), ((), ())),
                              preferred_element_type=_f32) + b_qkv[...]
        q_t.append(qkv[:, :D])
        k_t.append(qkv[:, D:2 * D])
        v_t.append(qkv[:, 2 * D:])
    K_all = jnp.concatenate(k_t, axis=1)
    V_all = jnp.concatenate(v_t, axis=1)

    sc_rows = []
    for i in range(T):
        qrep = jnp.dot(q_t[i], rep, preferred_element_type=_f32)
        sc_rows.append(jnp.dot(qrep * K_all, hm2,
                               preferred_element_type=_f32))
    SCw = jnp.concatenate(sc_rows, axis=1)
    Ew = jnp.exp(SCw)
    den = jnp.dot(Ew, g, preferred_element_type=_f32)
    den_b = jnp.dot(den, gt, preferred_element_type=_f32)
    Aw = Ew / den_b

    hacc = jnp.zeros((NB, D), _f32)
    for i in range(T):
        Ai = jnp.dot(Aw[:, i * T * H:(i + 1) * T * H], ex2,
                     preferred_element_type=_f32)
        att = jnp.dot(Ai * V_all, sumj, preferred_element_type=_f32)
        r = lax.dot_general(att, W_o[...], (((1,), (1,)), ((), ())),
                            preferred_element_type=_f32) + b_o[...] + xs_t[i]
        mu = r.mean(axis=-1, keepdims=True)
        var = ((r - mu) ** 2).mean(axis=-1, keepdims=True)
        hacc = hacc + (r - mu) * lax.rsqrt(var + 1e-5) * ln_g[...] + ln_b[...]
    hfeat = hacc / T

    hs, sss, sds = [], [], []
    for h in range(H):
        hh = jnp.dot(hfeat, W1[...][:, h * HF:(h + 1) * HF],
                     preferred_element_type=_f32)
        hs.append(hh)
        sss.append((hh * a_src[...][h]).sum(-1))
        sds.append((hh * a_dst[...][h]).sum(-1))
    z = [jnp.zeros((NB,), _f32)] * (16 - H)
    h1h_ref[...] = jnp.stack(hs, axis=0)
    ss_ref[...] = jnp.stack(sss + z, axis=1)
    sd_ref[...] = jnp.stack(sds + z, axis=1)


def _full(shape):
    nd = len(shape)
    return pl.BlockSpec(shape, lambda i: (0,) * nd)


_dense_call = pl.pallas_call(
    _dense_body,
    grid=(GRID,),
    in_specs=[
        pl.BlockSpec((T, NB, F_IN), lambda i: (0, i, 0)),
        _full((F_IN, D)), _full((D,)), _full((3 * D, D)), _full((3 * D,)),
        _full((D, D)), _full((D,)), _full((D,)), _full((D,)),
        _full((D, H * HF)), _full((H, HF)), _full((H, HF)),
        _full((D, T * D)), _full((T * D, T * H)), _full((T * T * H, T * H)),
        _full((T * H, T * T * H)), _full((T * H, T * D)), _full((T * D, D)),
    ],
    out_specs=[
        pl.BlockSpec((H, NB, HF), lambda i: (0, i, 0)),
        pl.BlockSpec((NB, 16), lambda i: (i, 0)),
        pl.BlockSpec((NB, 16), lambda i: (i, 0)),
    ],
    out_shape=[
        jax.ShapeDtypeStruct((H, N, HF), _f32),
        jax.ShapeDtypeStruct((N, 16), _f32),
        jax.ShapeDtypeStruct((N, 16), _f32),
    ],
)


def _mid_body(aggh_ref, den_ref, bias, W2, a_src, a_dst,
              h2h_ref, ss_ref, sd_ref):
    acc = jnp.zeros((NB, HF), _f32)
    for h in range(H):
        d = den_ref[...][:, h].reshape(NB, 1)
        acc = acc + aggh_ref[...][h] / (d + 1e-16)
    hfeat = jnp.maximum(acc / H + bias[...], 0.0)
    hs, sss, sds = [], [], []
    for h in range(H):
        hh = jnp.dot(hfeat, W2[...][:, h * HF:(h + 1) * HF],
                     preferred_element_type=_f32)
        hs.append(hh)
        sss.append((hh * a_src[...][h]).sum(-1))
        sds.append((hh * a_dst[...][h]).sum(-1))
    z = [jnp.zeros((NB,), _f32)] * (16 - H)
    h2h_ref[...] = jnp.stack(hs, axis=0)
    ss_ref[...] = jnp.stack(sss + z, axis=1)
    sd_ref[...] = jnp.stack(sds + z, axis=1)


_mid_call = pl.pallas_call(
    _mid_body,
    grid=(GRID,),
    in_specs=[
        pl.BlockSpec((H, NB, HF), lambda i: (0, i, 0)),
        pl.BlockSpec((NB, H), lambda i: (i, 0)),
        _full((HF,)), _full((HF, H * HF)), _full((H, HF)), _full((H, HF)),
    ],
    out_specs=[
        pl.BlockSpec((H, NB, HF), lambda i: (0, i, 0)),
        pl.BlockSpec((NB, 16), lambda i: (i, 0)),
        pl.BlockSpec((NB, 16), lambda i: (i, 0)),
    ],
    out_shape=[
        jax.ShapeDtypeStruct((H, N, HF), _f32),
        jax.ShapeDtypeStruct((N, 16), _f32),
        jax.ShapeDtypeStruct((N, 16), _f32),
    ],
)


def _final_body(aggh_ref, den_ref, bias, fc_W, fc_b, out_ref):
    acc = jnp.zeros((NB, HF), _f32)
    for h in range(H):
        d = den_ref[...][:, h].reshape(NB, 1)
        acc = acc + aggh_ref[...][h] / (d + 1e-16)
    hfeat = jnp.maximum(acc / H + bias[...], 0.0)
    out_ref[...] = jnp.dot(hfeat, fc_W[...],
                           preferred_element_type=_f32) + fc_b[...]


_final_call = pl.pallas_call(
    _final_body,
    grid=(GRID,),
    in_specs=[
        pl.BlockSpec((H, NB, HF), lambda i: (0, i, 0)),
        pl.BlockSpec((NB, H), lambda i: (i, 0)),
        _full((HF,)), _full((HF, 1)), _full((1,)),
    ],
    out_specs=pl.BlockSpec((NB, 1), lambda i: (i, 0)),
    out_shape=jax.ShapeDtypeStruct((N, 1), _f32),
)


BE = 6400


def _etrans_body(sel_r, ein_r, eout_r):
    eout_r[...] = lax.dot_general(sel_r[...], ein_r[...],
                                  (((1,), (1,)), ((), ())),
                                  preferred_element_type=_f32)


_etrans_call = pl.pallas_call(
    _etrans_body,
    grid=(E_PAD // BE,),
    in_specs=[
        _full((H, 16)),
        pl.BlockSpec((BE, 16), lambda i: (i, 0)),
    ],
    out_specs=pl.BlockSpec((H, BE), lambda i: (0, i)),
    out_shape=jax.ShapeDtypeStruct((H, E_PAD), _f32),
)

_SEL = np.eye(H, 16, dtype=np.float32)


def _escore_body(srcp, dstp, ss, sd, e1, sidx, didx, ssr, sdr, ew,
                 sl0, sl1, sa0, sa1, sb0, sb1, sw0, sw1):
    c = lax.axis_index("c")
    s = lax.axis_index("s")
    wid = s * NC + c
    cpt = E_PAD // (NW * CH)
    sem_l = (sl0, sl1)
    sem_a = (sa0, sa1)
    sem_b = (sb0, sb1)
    sem_w = (sw0, sw1)

    def issue_lin(i, b):
        base = (wid * cpt + i) * CH
        pltpu.async_copy(srcp.at[pl.ds(base, CH)], sidx.at[b], sem_l[b])
        pltpu.async_copy(dstp.at[pl.ds(base, CH)], didx.at[b], sem_l[b])

    def wait_lin(b):
        pltpu.make_async_copy(srcp.at[pl.ds(0, CH)], sidx.at[b],
                              sem_l[b]).wait()
        pltpu.make_async_copy(dstp.at[pl.ds(0, CH)], didx.at[b],
                              sem_l[b]).wait()

    def issue_gath(b):
        pltpu.async_copy(ss.at[sidx.at[b]], ssr.at[b], sem_a[b])
        pltpu.async_copy(sd.at[didx.at[b]], sdr.at[b], sem_b[b])

    def wait_gath(b):
        pltpu.make_async_copy(ss.at[sidx.at[b]], ssr.at[b], sem_a[b]).wait()
        pltpu.make_async_copy(sd.at[didx.at[b]], sdr.at[b], sem_b[b]).wait()

    def compute(b):
        def vec(j, carry2):
            al = ssr[b, j, pl.ds(0, LANES)] + sdr[b, j, pl.ds(0, LANES)]
            al = jnp.where(al > 0, al, 0.2 * al)
            ew[b, j, pl.ds(0, LANES)] = jnp.exp(al)
            return carry2

        lax.fori_loop(0, CH, vec, 0, unroll=8)

    def issue_wr(i, b):
        base = (wid * cpt + i) * CH
        pltpu.async_copy(ew.at[b], e1.at[pl.ds(base, CH)], sem_w[b])

    def drain_wr(b):
        pltpu.make_async_copy(e1.at[pl.ds(0, CH)], ew.at[b], sem_w[b]).wait()

    issue_lin(0, 0)
    wait_lin(0)
    issue_gath(0)
    issue_lin(1, 1)
    wait_gath(0)
    compute(0)
    issue_wr(0, 0)
    wait_lin(1)
    issue_gath(1)
    issue_lin(2, 0)
    wait_gath(1)
    compute(1)
    issue_wr(1, 1)
    wait_lin(0)
    issue_gath(0)

    def body(g, carry):
        i0 = 2 * g
        i1 = 2 * g + 1
        inext = jnp.minimum(2 * g + 2, cpt - 1)
        issue_lin(i1, 1)
        wait_gath(0)
        drain_wr(0)
        compute(0)
        issue_wr(i0, 0)
        wait_lin(1)
        issue_gath(1)
        issue_lin(inext, 0)
        wait_gath(1)
        drain_wr(1)
        compute(1)
        issue_wr(i1, 1)
        wait_lin(0)
        issue_gath(0)
        return carry

    lax.fori_loop(1, cpt // 2, body, 0)
    wait_gath(0)
    drain_wr(0)
    drain_wr(1)


@functools.cache
def _get_escore_call():
  return pl.kernel(
    _escore_body,
    out_type=jax.ShapeDtypeStruct((E_PAD, 16), _f32),
    mesh=plsc.VectorSubcoreMesh(core_axis_name="c", subcore_axis_name="s",
                                num_cores=NC, num_subcores=NS),
    scratch_types=[
        pltpu.VMEM((2, CH), jnp.int32),
        pltpu.VMEM((2, CH), jnp.int32),
        pltpu.VMEM((2, CH, 16), _f32),
        pltpu.VMEM((2, CH, 16), _f32),
        pltpu.VMEM((2, CH, 16), _f32),
        pltpu.SemaphoreType.DMA,
        pltpu.SemaphoreType.DMA,
        pltpu.SemaphoreType.DMA,
        pltpu.SemaphoreType.DMA,
        pltpu.SemaphoreType.DMA,
        pltpu.SemaphoreType.DMA,
        pltpu.SemaphoreType.DMA,
        pltpu.SemaphoreType.DMA,
    ],
    compiler_params=pltpu.CompilerParams(use_tc_tiling_on_sc=False),
  )


def _agg_body(srcp, dstp, e1, h1f, z2d, z1d, aggh, denp,
              sidx, didx, ech, gidx, rowsb, numacc, denacc,
              sl0, sl1, sg0, sg1, sn0, sn1, sd0, sd1):
    c = lax.axis_index("c")
    s = lax.axis_index("s")
    cpt = E_PAD // (NS * CH)
    sem_l = (sl0, sl1)
    sem_g = (sg0, sg1)
    sem_n = (sn0, sn1)
    sem_d = (sd0, sd1)

    for r in range(2):
        q = 2 * r + c
        qoff = q * N_PAD
        pltpu.sync_copy(z2d.at[pl.ds(0, RPT)], numacc.at[pl.ds(s * RPT, RPT)])
        pltpu.sync_copy(z1d.at[pl.ds(s * RPT, RPT)],
                        denacc.at[pl.ds(s * RPT, RPT)])
        plsc.subcore_barrier()

        def issue_lin(i, b):
            base = (s * cpt + i) * CH
            pltpu.async_copy(srcp.at[pl.ds(base, CH)], sidx.at[b], sem_l[b])
            pltpu.async_copy(dstp.at[pl.ds(base, CH)], didx.at[b], sem_l[b])
            pltpu.async_copy(e1.at[q, pl.ds(base, CH)],
                             ech.at[b, pl.ds(0, CH)], sem_l[b])

        def wait_lin(b):
            pltpu.make_async_copy(srcp.at[pl.ds(0, CH)], sidx.at[b],
                                  sem_l[b]).wait()
            pltpu.make_async_copy(dstp.at[pl.ds(0, CH)], didx.at[b],
                                  sem_l[b]).wait()
            pltpu.make_async_copy(e1.at[0, pl.ds(0, CH)],
                                  ech.at[b, pl.ds(0, CH)], sem_l[b]).wait()

        def prep_gather(b):
            for v in range(CH // LANES):
                gidx[b, pl.ds(v * LANES, LANES)] = (
                    sidx[b, pl.ds(v * LANES, LANES)] + qoff)
            pltpu.async_copy(h1f.at[gidx.at[b]], rowsb.at[b], sem_g[b])

        def wait_gather(b):
            pltpu.make_async_copy(h1f.at[gidx.at[b]], rowsb.at[b],
                                  sem_g[b]).wait()

        def scale_scatter(b):
            def scl(j, carry2):
                ev = ech[b, pl.ds(j, LANES)]
                e = ev[0]
                a = rowsb[b, j, pl.ds(0, LANES)]
                rowsb[b, j, pl.ds(0, LANES)] = a * e
                a2 = rowsb[b, j, pl.ds(LANES, LANES)]
                rowsb[b, j, pl.ds(LANES, LANES)] = a2 * e
                return carry2

            lax.fori_loop(0, CH, scl, 0, unroll=8)
            pltpu.async_copy(ech.at[b, pl.ds(0, CH)], denacc.at[didx.at[b]],
                             sem_d[b], add=True)
            pltpu.async_copy(rowsb.at[b], numacc.at[didx.at[b]],
                             sem_n[b], add=True)

        def drain_scat(b):
            pltpu.make_async_copy(ech.at[b, pl.ds(0, CH)],
                                  denacc.at[didx.at[b]], sem_d[b]).wait()
            pltpu.make_async_copy(rowsb.at[b], numacc.at[didx.at[b]],
                                  sem_n[b]).wait()

        issue_lin(0, 0)
        wait_lin(0)
        prep_gather(0)
        pltpu.async_copy(srcp.at[pl.ds(s * cpt * CH, CH)], sidx.at[1],
                         sem_l[1])
        pltpu.async_copy(dstp.at[pl.ds(s * cpt * CH, CH)], didx.at[1],
                         sem_l[1])
        pltpu.make_async_copy(srcp.at[pl.ds(0, CH)], sidx.at[1],
                              sem_l[1]).wait()
        pltpu.make_async_copy(dstp.at[pl.ds(0, CH)], didx.at[1],
                              sem_l[1]).wait()
        pltpu.sync_copy(z2d.at[pl.ds(0, CH)], rowsb.at[1])
        pltpu.sync_copy(z1d.at[pl.ds(0, CH)], ech.at[1, pl.ds(0, CH)])
        pltpu.async_copy(ech.at[1, pl.ds(0, CH)], denacc.at[didx.at[1]],
                         sem_d[1], add=True)
        pltpu.async_copy(rowsb.at[1], numacc.at[didx.at[1]],
                         sem_n[1], add=True)

        def body(g, carry):
            i1 = 2 * g + 1
            inext = jnp.minimum(2 * g + 2, cpt - 1)
            drain_scat(1)
            issue_lin(i1, 1)
            wait_gather(0)
            scale_scatter(0)
            wait_lin(1)
            prep_gather(1)
            drain_scat(0)
            issue_lin(inext, 0)
            wait_lin(0)
            wait_gather(1)
            scale_scatter(1)
            prep_gather(0)
            return carry

        lax.fori_loop(0, cpt // 2, body, 0)
        wait_gather(0)
        drain_scat(1)
        plsc.subcore_barrier()
        pltpu.sync_copy(numacc.at[pl.ds(s * RPT, RPT)],
                        aggh.at[q, pl.ds(s * RPT, RPT)])
        pltpu.sync_copy(denacc.at[pl.ds(s * RPT, RPT)],
                        denp.at[q, pl.ds(s * RPT, RPT)])
        plsc.subcore_barrier()


@functools.cache
def _get_agg_call():
  return pl.kernel(
    _agg_body,
    out_type=[
        jax.ShapeDtypeStruct((H, N_PAD, HF), _f32),
        jax.ShapeDtypeStruct((H, N_PAD), _f32),
    ],
    mesh=plsc.VectorSubcoreMesh(core_axis_name="c", subcore_axis_name="s",
                                num_cores=NC, num_subcores=NS),
    scratch_types=[
        pltpu.VMEM((2, CH), jnp.int32),
        pltpu.VMEM((2, CH), jnp.int32),
        pltpu.VMEM((2, CH + LANES), _f32),
        pltpu.VMEM((2, CH), jnp.int32),
        pltpu.VMEM((2, CH, HF), _f32),
        pltpu.VMEM_SHARED((N_PAD, HF), _f32),
        pltpu.VMEM_SHARED((N_PAD,), _f32),
        pltpu.SemaphoreType.DMA,
        pltpu.SemaphoreType.DMA,
        pltpu.SemaphoreType.DMA,
        pltpu.SemaphoreType.DMA,
        pltpu.SemaphoreType.DMA,
        pltpu.SemaphoreType.DMA,
        pltpu.SemaphoreType.DMA,
        pltpu.SemaphoreType.DMA,
    ],
    compiler_params=pltpu.CompilerParams(use_tc_tiling_on_sc=False),
  )


def kernel(x, edge_index, W_in, b_in, W_qkv, b_qkv, W_o, b_o, ln_g, ln_b,
           W1, att_src1, att_dst1, bias1, W2, att_src2, att_dst2, bias2,
           fc_W, fc_b):
    src = edge_index[0].astype(jnp.int32)
    dst = edge_index[1].astype(jnp.int32)
    pad = jnp.full((E_PAD - E0,), N, jnp.int32)
    srcp = jnp.concatenate([src, pad])
    dstp = jnp.concatenate([dst, pad])

    xT = x.reshape(N, T, F_IN).transpose(1, 0, 2)
    h1h, ss1, sd1 = _dense_call(xT, W_in, b_in, W_qkv, b_qkv, W_o, b_o,
                                ln_g, ln_b, W1, att_src1, att_dst1,
                                jnp.asarray(_REP), jnp.asarray(_HM2),
                                jnp.asarray(_G), jnp.asarray(_G.T),
                                jnp.asarray(_EX2), jnp.asarray(_SUMJ))

    z2d = jnp.zeros((RPT, HF), _f32)
    z1d = jnp.zeros((N_PAD,), _f32)

    def spad(a):
        return jnp.concatenate([a, jnp.full((N_PAD - N, 16), NEG, _f32)])

    def hpad(hh):
        return jnp.concatenate(
            [hh, jnp.zeros((H, N_PAD - N, HF), _f32)], axis=1
        ).reshape(H * N_PAD, HF)

    escore = _get_escore_call()
    agg = _get_agg_call()
    sel = jnp.asarray(_SEL)
    e1 = _etrans_call(sel, escore(srcp, dstp, spad(ss1), spad(sd1)))
    agg1, den1 = agg(srcp, dstp, e1, hpad(h1h), z2d, z1d)

    h2h, ss2, sd2 = _mid_call(agg1[:, :N], den1[:, :N].T, bias1, W2,
                              att_src2, att_dst2)
    e2 = _etrans_call(sel, escore(srcp, dstp, spad(ss2), spad(sd2)))
    agg2, den2 = agg(srcp, dstp, e2, hpad(h2h), z2d, z1d)

    return _final_call(agg2[:, :N], den2[:, :N].T, bias2, fc_W, fc_b)
